# baseline (speedup 1.0000x reference)
import functools

import jax
import jax.numpy as jnp
from jax import lax
from jax.experimental import pallas as pl
from jax.experimental.pallas import tpu as pltpu
from jax.experimental.pallas import tpu_sc as plsc

B, C, H, W = 4, 8, 1024, 1024
N_ERA = 65536
N = H * W
NCH = B * C
LANES = 16
CHUNK = 8192
CWORDS = CHUNK // 2
NCHUNK = N // CHUNK
SEG_SCALE = float(N_ERA) / float(N)
NC, NS = 2, 16


ROWS = CHUNK // W


def _sc_body(data_hbm, map_hbm, out_hbm,
             idx0, idx1, val0, val1, acc,
             sem_i0, sem_i1, sem_v0, sem_v1):
    wid = lax.axis_index("s") * NC + lax.axis_index("c")

    pltpu.async_copy(map_hbm.at[pl.ds(0, CWORDS)], idx0, sem_i0)
    pltpu.async_copy(data_hbm.at[wid, pl.ds(0, ROWS), :], val0, sem_v0)
    pltpu.async_copy(map_hbm.at[pl.ds(CWORDS, CWORDS)], idx1, sem_i1)
    pltpu.async_copy(data_hbm.at[wid, pl.ds(ROWS, ROWS), :], val1, sem_v1)

    zeros = jnp.zeros((LANES,), jnp.float32)

    def zero_body(i, carry):
        acc[pl.ds(i * LANES, LANES)] = zeros
        return carry

    lax.fori_loop(0, N_ERA // LANES, zero_body, 0, unroll=8)

    def scatter_chunk(idx_buf, val_buf):
        @plsc.parallel_loop(0, CHUNK // (2 * LANES), unroll=8)
        def _(j):
            r = j // (W // (2 * LANES))
            c = (j % (W // (2 * LANES))) * (2 * LANES)
            packed = idx_buf[pl.ds(j * LANES, LANES)]
            ia = packed & 0xFFFF
            ib = lax.shift_right_logical(packed, 16)
            va = val_buf[r, pl.ds(c, LANES)] * SEG_SCALE
            vb = val_buf[r, pl.ds(c + LANES, LANES)] * SEG_SCALE
            plsc.addupdate_scatter(acc, [ia], va)
            plsc.addupdate_scatter(acc, [ib], vb)

    def chunk_pair(gp, carry):
        g0 = gp * 2
        pltpu.make_async_copy(map_hbm.at[pl.ds(0, CWORDS)], idx0, sem_i0).wait()
        pltpu.make_async_copy(data_hbm.at[0, pl.ds(0, ROWS), :], val0, sem_v0).wait()
        scatter_chunk(idx0, val0)

        @pl.when(g0 + 2 < NCHUNK)
        def _():
            g = g0 + 2
            pltpu.async_copy(map_hbm.at[pl.ds(g * CWORDS, CWORDS)], idx0, sem_i0)
            pltpu.async_copy(data_hbm.at[wid, pl.ds(g * ROWS, ROWS), :], val0, sem_v0)

        pltpu.make_async_copy(map_hbm.at[pl.ds(0, CWORDS)], idx1, sem_i1).wait()
        pltpu.make_async_copy(data_hbm.at[0, pl.ds(0, ROWS), :], val1, sem_v1).wait()
        scatter_chunk(idx1, val1)

        @pl.when(g0 + 3 < NCHUNK)
        def _():
            g = g0 + 3
            pltpu.async_copy(map_hbm.at[pl.ds(g * CWORDS, CWORDS)], idx1, sem_i1)
            pltpu.async_copy(data_hbm.at[wid, pl.ds(g * ROWS, ROWS), :], val1, sem_v1)

        return carry

    lax.fori_loop(0, NCHUNK // 2, chunk_pair, 0)

    pltpu.sync_copy(acc, out_hbm.at[pl.ds(wid * N_ERA, N_ERA)])


@jax.jit
def _mean_to_era5(data3, mapping):
    mesh = plsc.VectorSubcoreMesh(
        core_axis_name="c", subcore_axis_name="s", num_cores=NC, num_subcores=NS)
    return pl.kernel(
        _sc_body,
        out_type=jax.ShapeDtypeStruct((NCH * N_ERA,), jnp.float32),
        mesh=mesh,
        compiler_params=pltpu.CompilerParams(needs_layout_passes=False),
        scratch_types=[
            pltpu.VMEM((CWORDS,), jnp.int32),
            pltpu.VMEM((CWORDS,), jnp.int32),
            pltpu.VMEM((ROWS, W), jnp.float32),
            pltpu.VMEM((ROWS, W), jnp.float32),
            pltpu.VMEM((N_ERA,), jnp.float32),
            pltpu.SemaphoreType.DMA,
            pltpu.SemaphoreType.DMA,
            pltpu.SemaphoreType.DMA,
            pltpu.SemaphoreType.DMA,
        ],
    )(data3, mapping)


def kernel(output, mapping):
    data3 = output.reshape(NCH, H, W)
    m16 = mapping.astype(jnp.uint16)
    m16 = m16.reshape(N // 32, 2, 16).swapaxes(1, 2).reshape(N // 2, 2)
    m32 = lax.bitcast_convert_type(m16, jnp.int32)
    out_flat = _mean_to_era5(data3, m32)
    return out_flat.reshape(B, C, N_ERA)


# speedup vs baseline: 2.4716x; 2.4716x over previous
import jax
import jax.numpy as jnp
from jax import lax
from jax.experimental import pallas as pl
from jax.experimental.pallas import tpu as pltpu
from jax.experimental.pallas import tpu_sc as plsc

B, C, H, W = 4, 8, 1024, 1024
N_ERA = 65536
N = H * W
NCH = B * C
LANES = 16
CHUNK = 8192
NCHUNK = N // 2 // CHUNK
ROWS = CHUNK // W
HROW = H // 2
SEG_SCALE = float(N_ERA) / float(N)
NC, NS = 2, 16


def _sc_body(data_hbm, map_hbm, out_hbm,
             idx0, idx1, vlo0, vlo1, vhi0, vhi1, acc,
             sem_i0, sem_i1, sem_l0, sem_l1, sem_h0, sem_h1):
    wid = lax.axis_index("s") * NC + lax.axis_index("c")

    def fill(g, idx_buf, vlo_buf, vhi_buf, sem_i, sem_l, sem_h):
        pltpu.async_copy(map_hbm.at[pl.ds(g * CHUNK, CHUNK)], idx_buf, sem_i)
        pltpu.async_copy(
            data_hbm.at[wid, pl.ds(g * ROWS, ROWS), :], vlo_buf, sem_l)
        pltpu.async_copy(
            data_hbm.at[wid, pl.ds(HROW + g * ROWS, ROWS), :], vhi_buf, sem_h)

    def wait(idx_buf, vlo_buf, vhi_buf, sem_i, sem_l, sem_h):
        pltpu.make_async_copy(map_hbm.at[pl.ds(0, CHUNK)], idx_buf, sem_i).wait()
        pltpu.make_async_copy(
            data_hbm.at[0, pl.ds(0, ROWS), :], vlo_buf, sem_l).wait()
        pltpu.make_async_copy(
            data_hbm.at[0, pl.ds(0, ROWS), :], vhi_buf, sem_h).wait()

    fill(0, idx0, vlo0, vhi0, sem_i0, sem_l0, sem_h0)
    fill(1, idx1, vlo1, vhi1, sem_i1, sem_l1, sem_h1)

    zeros = jnp.zeros((LANES,), jnp.float32)

    def zero_body(i, carry):
        acc[pl.ds(i * LANES, LANES)] = zeros
        return carry

    lax.fori_loop(0, N_ERA // LANES, zero_body, 0, unroll=8)

    def scatter_chunk(idx_buf, vlo_buf, vhi_buf):
        @plsc.parallel_loop(0, CHUNK // LANES, unroll=8)
        def _(j):
            r = j // (W // LANES)
            c = (j % (W // LANES)) * LANES
            packed = idx_buf[pl.ds(j * LANES, LANES)]
            ia = packed & 0xFFFF
            ib = lax.shift_right_logical(packed, 16)
            va = vlo_buf[r, pl.ds(c, LANES)] * SEG_SCALE
            vb = vhi_buf[r, pl.ds(c, LANES)] * SEG_SCALE
            plsc.addupdate_scatter(acc, [ia], va)
            plsc.addupdate_scatter(acc, [ib], vb)

    def chunk_pair(gp, carry):
        g0 = gp * 2
        wait(idx0, vlo0, vhi0, sem_i0, sem_l0, sem_h0)
        scatter_chunk(idx0, vlo0, vhi0)

        @pl.when(g0 + 2 < NCHUNK)
        def _():
            fill(g0 + 2, idx0, vlo0, vhi0, sem_i0, sem_l0, sem_h0)

        wait(idx1, vlo1, vhi1, sem_i1, sem_l1, sem_h1)
        scatter_chunk(idx1, vlo1, vhi1)

        @pl.when(g0 + 3 < NCHUNK)
        def _():
            fill(g0 + 3, idx1, vlo1, vhi1, sem_i1, sem_l1, sem_h1)

        return carry

    lax.fori_loop(0, NCHUNK // 2, chunk_pair, 0)

    pltpu.sync_copy(acc, out_hbm.at[pl.ds(wid * N_ERA, N_ERA)])


@jax.jit
def _mean_to_era5(data3, packed_map):
    mesh = plsc.VectorSubcoreMesh(
        core_axis_name="c", subcore_axis_name="s", num_cores=NC, num_subcores=NS)
    return pl.kernel(
        _sc_body,
        out_type=jax.ShapeDtypeStruct((NCH * N_ERA,), jnp.float32),
        mesh=mesh,
        compiler_params=pltpu.CompilerParams(needs_layout_passes=False),
        scratch_types=[
            pltpu.VMEM((CHUNK,), jnp.int32),
            pltpu.VMEM((CHUNK,), jnp.int32),
            pltpu.VMEM((ROWS, W), jnp.float32),
            pltpu.VMEM((ROWS, W), jnp.float32),
            pltpu.VMEM((ROWS, W), jnp.float32),
            pltpu.VMEM((ROWS, W), jnp.float32),
            pltpu.VMEM((N_ERA,), jnp.float32),
            pltpu.SemaphoreType.DMA,
            pltpu.SemaphoreType.DMA,
            pltpu.SemaphoreType.DMA,
            pltpu.SemaphoreType.DMA,
            pltpu.SemaphoreType.DMA,
            pltpu.SemaphoreType.DMA,
        ],
    )(data3, packed_map)


def kernel(output, mapping):
    data3 = output.reshape(NCH, H, W)
    m32 = mapping[: N // 2] | (mapping[N // 2:] << 16)
    out_flat = _mean_to_era5(data3, m32)
    return out_flat.reshape(B, C, N_ERA)


# speedup vs baseline: 2.4953x; 1.0096x over previous
import jax
import jax.numpy as jnp
from jax import lax
from jax.experimental import pallas as pl
from jax.experimental.pallas import tpu as pltpu
from jax.experimental.pallas import tpu_sc as plsc

B, C, H, W = 4, 8, 1024, 1024
N_ERA = 65536
N = H * W
NCH = B * C
LANES = 16
CHUNK = 4096
NCHUNK = N // 2 // CHUNK
ROWS = CHUNK // W
HROW = H // 2
SEG_SCALE = float(N_ERA) / float(N)
NC, NS = 2, 16
NBUF = 4


def _sc_body(data_hbm, map_hbm, out_hbm, *refs):
    idxs = refs[0:NBUF]
    vlos = refs[NBUF:2 * NBUF]
    vhis = refs[2 * NBUF:3 * NBUF]
    acc = refs[3 * NBUF]
    sems_i = refs[3 * NBUF + 1:3 * NBUF + 1 + NBUF]
    sems_l = refs[3 * NBUF + 1 + NBUF:3 * NBUF + 1 + 2 * NBUF]
    sems_h = refs[3 * NBUF + 1 + 2 * NBUF:3 * NBUF + 1 + 3 * NBUF]
    wid = lax.axis_index("s") * NC + lax.axis_index("c")

    def fill(g, b):
        pltpu.async_copy(map_hbm.at[pl.ds(g * CHUNK, CHUNK)], idxs[b], sems_i[b])
        pltpu.async_copy(
            data_hbm.at[wid, pl.ds(g * ROWS, ROWS), :], vlos[b], sems_l[b])
        pltpu.async_copy(
            data_hbm.at[wid, pl.ds(HROW + g * ROWS, ROWS), :], vhis[b], sems_h[b])

    def wait(b):
        pltpu.make_async_copy(
            map_hbm.at[pl.ds(0, CHUNK)], idxs[b], sems_i[b]).wait()
        pltpu.make_async_copy(
            data_hbm.at[0, pl.ds(0, ROWS), :], vlos[b], sems_l[b]).wait()
        pltpu.make_async_copy(
            data_hbm.at[0, pl.ds(0, ROWS), :], vhis[b], sems_h[b]).wait()

    for b in range(NBUF):
        fill(b, b)

    zeros = jnp.zeros((LANES,), jnp.float32)

    def zero_body(i, carry):
        acc[pl.ds(i * LANES, LANES)] = zeros
        return carry

    lax.fori_loop(0, N_ERA // LANES, zero_body, 0, unroll=8)

    def scatter_chunk(b):
        idx_buf, vlo_buf, vhi_buf = idxs[b], vlos[b], vhis[b]
        @plsc.parallel_loop(0, CHUNK // LANES, unroll=8)
        def _(j):
            r = j // (W // LANES)
            c = (j % (W // LANES)) * LANES
            packed = idx_buf[pl.ds(j * LANES, LANES)]
            ia = packed & 0xFFFF
            ib = lax.shift_right_logical(packed, 16)
            va = vlo_buf[r, pl.ds(c, LANES)] * SEG_SCALE
            vb = vhi_buf[r, pl.ds(c, LANES)] * SEG_SCALE
            plsc.addupdate_scatter(acc, [ia], va)
            plsc.addupdate_scatter(acc, [ib], vb)

    def chunk_group(gp, carry):
        g0 = gp * NBUF
        for b in range(NBUF):
            wait(b)
            scatter_chunk(b)

            @pl.when(g0 + b + NBUF < NCHUNK)
            def _():
                fill(g0 + b + NBUF, b)

        return carry

    lax.fori_loop(0, NCHUNK // NBUF, chunk_group, 0)

    pltpu.sync_copy(acc, out_hbm.at[pl.ds(wid * N_ERA, N_ERA)])


@jax.jit
def _mean_to_era5(data3, packed_map):
    mesh = plsc.VectorSubcoreMesh(
        core_axis_name="c", subcore_axis_name="s", num_cores=NC, num_subcores=NS)
    return pl.kernel(
        _sc_body,
        out_type=jax.ShapeDtypeStruct((NCH * N_ERA,), jnp.float32),
        mesh=mesh,
        compiler_params=pltpu.CompilerParams(needs_layout_passes=False),
        scratch_types=(
            [pltpu.VMEM((CHUNK,), jnp.int32)] * NBUF
            + [pltpu.VMEM((ROWS, W), jnp.float32)] * (2 * NBUF)
            + [pltpu.VMEM((N_ERA,), jnp.float32)]
            + [pltpu.SemaphoreType.DMA] * (3 * NBUF)
        ),
    )(data3, packed_map)


def kernel(output, mapping):
    data3 = output.reshape(NCH, H, W)
    m32 = mapping[: N // 2] | (mapping[N // 2:] << 16)
    out_flat = _mean_to_era5(data3, m32)
    return out_flat.reshape(B, C, N_ERA)
